# Initial kernel scaffold; baseline (speedup 1.0000x reference)
import jax
import jax.numpy as jnp
from jax.experimental import pallas as pl


def kernel(x, edge_index, W, att_src, att_dst, bias):
    raise NotImplementedError("write your pallas kernel here")


# speedup vs baseline: 12.0310x; 12.0310x over previous
import jax
import jax.numpy as jnp
from jax import lax
from jax.experimental import pallas as pl
from jax.experimental.pallas import tpu as pltpu
from jax.experimental.pallas import tpu_sc as plsc

N = 50000
E = 800000
FEAT = 128

N_PAD = 51200
CHUNK = 5120
CPS = 5
TILES = 16
ROWS_PER_TILE = CHUNK // TILES
STG = 64
EPT = E // TILES
SLAB = 2000
NSLAB = EPT // SLAB
GRP = SLAB // 16
GB = 80
NB_MAX = SLAB // GB

_BM1 = 2000
_BM3 = 800


def _phase1_body(x_ref, w_ref, att_ref, h_ref, a_ref):
    h = jnp.dot(x_ref[...], w_ref[...], preferred_element_type=jnp.float32)
    h_ref[...] = h
    a_ref[...] = jnp.dot(h, att_ref[...], preferred_element_type=jnp.float32)


def _phase3_body(acc_ref, ws_ref, b_ref, o_ref):
    o_ref[...] = jnp.tanh(acc_ref[...] / (ws_ref[...] + 1e-16) + b_ref[...])


def _edge_body(hmat, asrc, adst, es, ed, acc_out, wsum_out,
               asrc_v, adst_v, s_buf, d_buf, sc_buf, wc_buf, dc_buf,
               rows_v, stage, wstage, accum, wacc, sem):
    c = lax.axis_index("c")
    s = lax.axis_index("s")
    ebase = s * EPT
    lane = lax.iota(jnp.int32, 16)
    zf16 = jnp.zeros((16,), jnp.float32)
    zi16 = jnp.zeros((16,), jnp.int32)

    pltpu.sync_copy(asrc, asrc_v)

    def init_g(g, carry):
        sc_buf[pl.ds(g * 16, 16)] = zi16
        wc_buf[pl.ds(g * 16, 16)] = zf16
        dc_buf[g // 5, pl.ds((g % 5) * 16, 16)] = lane + g * 16
        return carry
    lax.fori_loop(0, GRP, init_g, 0)
    for k in range(7):
        wc_buf[pl.ds(SLAB + 16 * k, 16)] = zf16

    def chunk_body(ci, carry):
        lo = (c * CPS + ci) * CHUNK

        def zg(g, cc):
            stage[g // 8, pl.ds((g % 8) * 16, 16)] = zf16
            return cc
        lax.fori_loop(0, STG * 8, zg, 0)

        def zw(g, cc):
            wstage[pl.ds(g * 16, 16)] = zf16
            return cc
        lax.fori_loop(0, ROWS_PER_TILE // 16, zw, 0)

        for j in range(ROWS_PER_TILE // STG):
            pltpu.sync_copy(stage, accum.at[pl.ds(s * ROWS_PER_TILE + j * STG, STG)])
        pltpu.sync_copy(wstage, wacc.at[pl.ds(s * ROWS_PER_TILE, ROWS_PER_TILE)])
        pltpu.sync_copy(adst.at[pl.ds(lo, CHUNK)], adst_v)
        plsc.subcore_barrier()

        def slab_body(sl, cc):
            off0 = ebase + sl * SLAB
            pltpu.sync_copy(es.at[pl.ds(off0, SLAB)], s_buf)
            pltpu.sync_copy(ed.at[pl.ds(off0, SLAB)], d_buf)

            def grp_body(g, off_s):
                s16 = s_buf[pl.ds(g * 16, 16)]
                d16 = d_buf[pl.ds(g * 16, 16)]
                dl = d16 - lo
                inm = (dl >= 0) & (dl < CHUNK)
                dls = jnp.where(inm, dl, 0)
                a1 = plsc.load_gather(asrc_v, [s16])
                a2 = plsc.load_gather(adst_v, [dls])
                al = a1 + a2
                al = jnp.maximum(al, 0.2 * al)
                w = jnp.exp(al)
                mi = inm.astype(jnp.int32)
                pos = off_s + plsc.cumsum(mi) - 1
                plsc.store_scatter(sc_buf, [pos], s16, mask=inm)
                plsc.store_scatter(wc_buf, [pos], w, mask=inm)
                plsc.store_scatter(dc_buf, [pos // GB, pos % GB], dls, mask=inm)
                return off_s + jnp.sum(mi)

            cnt = lax.fori_loop(0, GRP, grp_body, jnp.int32(0))

            f16 = (cnt // 16) * 16
            rem = cnt - f16
            wv = wc_buf[pl.ds(f16, 16)]
            wc_buf[pl.ds(f16, 16)] = jnp.where(lane >= rem, 0.0, wv)
            for k in range(1, 7):
                wc_buf[pl.ds(f16 + 16 * k, 16)] = zf16

            nb = (cnt + GB - 1) // GB

            def flush_body(b, cc):
                boff = b * GB
                pltpu.async_copy(hmat.at[sc_buf.at[pl.ds(boff, GB)]],
                                 rows_v, sem).wait()
                for r16 in range(GB // 16):
                    wg = wc_buf[pl.ds(boff + r16 * 16, 16)]
                    for j in range(16):
                        r = r16 * 16 + j
                        ws = wg[j]
                        for v in range(8):
                            rows_v[r, pl.ds(v * 16, 16)] = rows_v[r, pl.ds(v * 16, 16)] * ws
                pltpu.sync_copy(rows_v, accum.at[dc_buf.at[b]], add=True)
                pltpu.sync_copy(wc_buf.at[pl.ds(boff, GB)],
                                wacc.at[dc_buf.at[b]], add=True)
                return cc
            lax.fori_loop(0, nb, flush_body, 0)
            return cc
        lax.fori_loop(0, NSLAB, slab_body, 0)

        plsc.subcore_barrier()
        for j in range(ROWS_PER_TILE // STG):
            rbase = s * ROWS_PER_TILE + j * STG
            pltpu.sync_copy(accum.at[pl.ds(rbase, STG)], stage)
            pltpu.sync_copy(stage, acc_out.at[pl.ds(lo + rbase, STG)])
        rbase = s * ROWS_PER_TILE
        pltpu.sync_copy(wacc.at[pl.ds(rbase, ROWS_PER_TILE)], wstage)
        pltpu.sync_copy(wstage, wsum_out.at[pl.ds(lo + rbase, ROWS_PER_TILE)])
        plsc.subcore_barrier()
        return carry
    lax.fori_loop(0, CPS, chunk_body, 0)


_edge_kernel = pl.kernel(
    _edge_body,
    mesh=plsc.VectorSubcoreMesh(core_axis_name="c", subcore_axis_name="s"),
    compiler_params=pltpu.CompilerParams(needs_layout_passes=False),
    out_type=[
        jax.ShapeDtypeStruct((N_PAD, FEAT), jnp.float32),
        jax.ShapeDtypeStruct((N_PAD,), jnp.float32),
    ],
    scratch_types=[
        pltpu.VMEM((N,), jnp.float32),
        pltpu.VMEM((CHUNK,), jnp.float32),
        pltpu.VMEM((SLAB,), jnp.int32),
        pltpu.VMEM((SLAB,), jnp.int32),
        pltpu.VMEM((SLAB,), jnp.int32),
        pltpu.VMEM((SLAB + 112,), jnp.float32),
        pltpu.VMEM((NB_MAX, GB), jnp.int32),
        pltpu.VMEM((GB, FEAT), jnp.float32),
        pltpu.VMEM((STG, FEAT), jnp.float32),
        pltpu.VMEM((ROWS_PER_TILE,), jnp.float32),
        pltpu.VMEM_SHARED((CHUNK, FEAT), jnp.float32),
        pltpu.VMEM_SHARED((CHUNK,), jnp.float32),
        pltpu.SemaphoreType.DMA,
    ],
)


@jax.jit
def kernel(x, edge_index, W, att_src, att_dst, bias):
    att = jnp.zeros((FEAT, 8), jnp.float32)
    att = att.at[:, 0].set(att_src[0]).at[:, 1].set(att_dst[0])

    h, a_out = pl.pallas_call(
        _phase1_body,
        grid=(N // _BM1,),
        in_specs=[
            pl.BlockSpec((_BM1, 4), lambda i: (i, 0)),
            pl.BlockSpec((4, FEAT), lambda i: (0, 0)),
            pl.BlockSpec((FEAT, 8), lambda i: (0, 0)),
        ],
        out_specs=[
            pl.BlockSpec((_BM1, FEAT), lambda i: (i, 0)),
            pl.BlockSpec((_BM1, 8), lambda i: (i, 0)),
        ],
        out_shape=[
            jax.ShapeDtypeStruct((N, FEAT), jnp.float32),
            jax.ShapeDtypeStruct((N, 8), jnp.float32),
        ],
    )(x, W, att)

    a_src = a_out[:, 0]
    a_dst = jnp.pad(a_out[:, 1], (0, N_PAD - N))
    es = edge_index[0]
    ed = edge_index[1]

    acc, wsum = _edge_kernel(h, a_src, a_dst, es, ed)

    out = pl.pallas_call(
        _phase3_body,
        grid=(N_PAD // _BM3,),
        in_specs=[
            pl.BlockSpec((_BM3, FEAT), lambda i: (i, 0)),
            pl.BlockSpec((_BM3, 1), lambda i: (i, 0)),
            pl.BlockSpec((1, FEAT), lambda i: (0, 0)),
        ],
        out_specs=pl.BlockSpec((_BM3, FEAT), lambda i: (i, 0)),
        out_shape=jax.ShapeDtypeStruct((N_PAD, FEAT), jnp.float32),
    )(acc, wsum.reshape(N_PAD, 1), bias.reshape(1, FEAT))

    return out[:N].ravel()


# speedup vs baseline: 12.0353x; 1.0004x over previous
import jax
import jax.numpy as jnp
from jax import lax
from jax.experimental import pallas as pl
from jax.experimental.pallas import tpu as pltpu
from jax.experimental.pallas import tpu_sc as plsc

N = 50000
E = 800000
FEAT = 128

N_PAD = 51200
CHUNK = 5120
CPS = 5
TILES = 16
ROWS_PER_TILE = CHUNK // TILES
STG = 64
EPT = E // TILES
SLAB = 2000
NSLAB = EPT // SLAB
GRP = SLAB // 16
GB = 80
NB_MAX = SLAB // GB

_BM1 = 2000
_BM3 = 800


def _phase1_body(x_ref, w_ref, att_ref, h_ref, a_ref):
    h = jnp.dot(x_ref[...], w_ref[...], preferred_element_type=jnp.float32)
    h_ref[...] = h
    a_ref[...] = jnp.dot(h, att_ref[...], preferred_element_type=jnp.float32)


def _phase3_body(acc_ref, ws_ref, b_ref, o_ref):
    o_ref[...] = jnp.tanh(acc_ref[...] / (ws_ref[...] + 1e-16) + b_ref[...])


def _edge_body(hmat, asrc, adst, es, ed, acc_out, wsum_out,
               asrc_v, adst_v, s_buf, d_buf, sc_buf, wc_buf, dc_buf,
               rows_v, stage, wstage, accum, wacc, sem):
    c = lax.axis_index("c")
    s = lax.axis_index("s")
    ebase = s * EPT
    lane = lax.iota(jnp.int32, 16)
    zf16 = jnp.zeros((16,), jnp.float32)
    zi16 = jnp.zeros((16,), jnp.int32)

    pltpu.sync_copy(asrc, asrc_v)

    def init_g(g, carry):
        sc_buf[pl.ds(g * 16, 16)] = zi16
        wc_buf[pl.ds(g * 16, 16)] = zf16
        dc_buf[g // 5, pl.ds((g % 5) * 16, 16)] = lane + g * 16
        return carry
    lax.fori_loop(0, GRP, init_g, 0)
    for k in range(7):
        wc_buf[pl.ds(SLAB + 16 * k, 16)] = zf16

    def chunk_body(ci, carry):
        lo = (c * CPS + ci) * CHUNK

        def zg(g, cc):
            stage[g // 8, pl.ds((g % 8) * 16, 16)] = zf16
            return cc
        lax.fori_loop(0, STG * 8, zg, 0)

        def zw(g, cc):
            wstage[pl.ds(g * 16, 16)] = zf16
            return cc
        lax.fori_loop(0, ROWS_PER_TILE // 16, zw, 0)

        for j in range(ROWS_PER_TILE // STG):
            pltpu.sync_copy(stage, accum.at[pl.ds(s * ROWS_PER_TILE + j * STG, STG)])
        pltpu.sync_copy(wstage, wacc.at[pl.ds(s * ROWS_PER_TILE, ROWS_PER_TILE)])
        pltpu.sync_copy(adst.at[pl.ds(lo, CHUNK)], adst_v)
        plsc.subcore_barrier()

        def slab_body(sl, cc):
            off0 = ebase + sl * SLAB
            pltpu.sync_copy(es.at[pl.ds(off0, SLAB)], s_buf)
            pltpu.sync_copy(ed.at[pl.ds(off0, SLAB)], d_buf)

            def grp_body(g, off_s):
                s16 = s_buf[pl.ds(g * 16, 16)]
                d16 = d_buf[pl.ds(g * 16, 16)]
                dl = d16 - lo
                inm = (dl >= 0) & (dl < CHUNK)
                dls = jnp.where(inm, dl, 0)
                a1 = plsc.load_gather(asrc_v, [s16])
                a2 = plsc.load_gather(adst_v, [dls])
                al = a1 + a2
                al = jnp.maximum(al, 0.2 * al)
                w = jnp.exp(al)
                mi = inm.astype(jnp.int32)
                pos = off_s + plsc.cumsum(mi) - 1
                plsc.store_scatter(sc_buf, [pos], s16, mask=inm)
                plsc.store_scatter(wc_buf, [pos], w, mask=inm)
                plsc.store_scatter(dc_buf, [pos // GB, pos % GB], dls, mask=inm)
                return off_s + plsc.all_reduce_population_count(inm)[0]

            cnt = lax.fori_loop(0, GRP, grp_body, jnp.int32(0))

            f16 = (cnt // 16) * 16
            rem = cnt - f16
            wv = wc_buf[pl.ds(f16, 16)]
            wc_buf[pl.ds(f16, 16)] = jnp.where(lane >= rem, 0.0, wv)
            for k in range(1, 7):
                wc_buf[pl.ds(f16 + 16 * k, 16)] = zf16

            nb = (cnt + GB - 1) // GB

            def flush_body(b, cc):
                boff = b * GB
                pltpu.async_copy(hmat.at[sc_buf.at[pl.ds(boff, GB)]],
                                 rows_v, sem).wait()
                for r16 in range(GB // 16):
                    wg = wc_buf[pl.ds(boff + r16 * 16, 16)]
                    for j in range(16):
                        r = r16 * 16 + j
                        ws = wg[j]
                        for v in range(8):
                            rows_v[r, pl.ds(v * 16, 16)] = (
                                rows_v[r, pl.ds(v * 16, 16)] * ws)
                pltpu.sync_copy(rows_v, accum.at[dc_buf.at[b]], add=True)
                pltpu.sync_copy(wc_buf.at[pl.ds(boff, GB)],
                                wacc.at[dc_buf.at[b]], add=True)
                return cc
            lax.fori_loop(0, nb, flush_body, 0)
            return cc
        lax.fori_loop(0, NSLAB, slab_body, 0)

        plsc.subcore_barrier()
        for j in range(ROWS_PER_TILE // STG):
            rbase = s * ROWS_PER_TILE + j * STG
            pltpu.sync_copy(accum.at[pl.ds(rbase, STG)], stage)
            pltpu.sync_copy(stage, acc_out.at[pl.ds(lo + rbase, STG)])
        rbase = s * ROWS_PER_TILE
        pltpu.sync_copy(wacc.at[pl.ds(rbase, ROWS_PER_TILE)], wstage)
        pltpu.sync_copy(wstage, wsum_out.at[pl.ds(lo + rbase, ROWS_PER_TILE)])
        plsc.subcore_barrier()
        return carry
    lax.fori_loop(0, CPS, chunk_body, 0)


_edge_kernel = pl.kernel(
    _edge_body,
    mesh=plsc.VectorSubcoreMesh(core_axis_name="c", subcore_axis_name="s"),
    compiler_params=pltpu.CompilerParams(needs_layout_passes=False),
    out_type=[
        jax.ShapeDtypeStruct((N_PAD, FEAT), jnp.float32),
        jax.ShapeDtypeStruct((N_PAD,), jnp.float32),
    ],
    scratch_types=[
        pltpu.VMEM((N,), jnp.float32),
        pltpu.VMEM((CHUNK,), jnp.float32),
        pltpu.VMEM((SLAB,), jnp.int32),
        pltpu.VMEM((SLAB,), jnp.int32),
        pltpu.VMEM((SLAB,), jnp.int32),
        pltpu.VMEM((SLAB + 112,), jnp.float32),
        pltpu.VMEM((NB_MAX, GB), jnp.int32),
        pltpu.VMEM((GB, FEAT), jnp.float32),
        pltpu.VMEM((STG, FEAT), jnp.float32),
        pltpu.VMEM((ROWS_PER_TILE,), jnp.float32),
        pltpu.VMEM_SHARED((CHUNK, FEAT), jnp.float32),
        pltpu.VMEM_SHARED((CHUNK,), jnp.float32),
        pltpu.SemaphoreType.DMA,
    ],
)


@jax.jit
def kernel(x, edge_index, W, att_src, att_dst, bias):
    att = jnp.zeros((FEAT, 8), jnp.float32)
    att = att.at[:, 0].set(att_src[0]).at[:, 1].set(att_dst[0])

    h, a_out = pl.pallas_call(
        _phase1_body,
        grid=(N // _BM1,),
        in_specs=[
            pl.BlockSpec((_BM1, 4), lambda i: (i, 0)),
            pl.BlockSpec((4, FEAT), lambda i: (0, 0)),
            pl.BlockSpec((FEAT, 8), lambda i: (0, 0)),
        ],
        out_specs=[
            pl.BlockSpec((_BM1, FEAT), lambda i: (i, 0)),
            pl.BlockSpec((_BM1, 8), lambda i: (i, 0)),
        ],
        out_shape=[
            jax.ShapeDtypeStruct((N, FEAT), jnp.float32),
            jax.ShapeDtypeStruct((N, 8), jnp.float32),
        ],
    )(x, W, att)

    a_src = a_out[:, 0]
    a_dst = jnp.pad(a_out[:, 1], (0, N_PAD - N))
    es = edge_index[0]
    ed = edge_index[1]

    acc, wsum = _edge_kernel(h, a_src, a_dst, es, ed)

    out = pl.pallas_call(
        _phase3_body,
        grid=(N_PAD // _BM3,),
        in_specs=[
            pl.BlockSpec((_BM3, FEAT), lambda i: (i, 0)),
            pl.BlockSpec((_BM3, 1), lambda i: (i, 0)),
            pl.BlockSpec((1, FEAT), lambda i: (0, 0)),
        ],
        out_specs=pl.BlockSpec((_BM3, FEAT), lambda i: (i, 0)),
        out_shape=jax.ShapeDtypeStruct((N_PAD, FEAT), jnp.float32),
    )(acc, wsum.reshape(N_PAD, 1), bias.reshape(1, FEAT))

    return out[:N].ravel()
